# Initial kernel scaffold; baseline (speedup 1.0000x reference)
import jax
import jax.numpy as jnp
from jax.experimental import pallas as pl


def kernel(x, Wg, bg, W1, b1, W2, b2):
    raise NotImplementedError("write your pallas kernel here")


# speedup vs baseline: 1.2052x; 1.2052x over previous
import functools

import jax
import jax.numpy as jnp
from jax import lax
from jax.experimental import pallas as pl
from jax.experimental.pallas import tpu as pltpu
from jax.experimental.pallas import tpu_sc as plsc

T = 2048
D = 768
E = 8
K = 2
DFF = 2048
OUT = 768

B = 256
NMAX = T * K + E * B
NB = NMAX // B

NC, NS = 2, 16
NW = NC * NS
LANES = 128


def _gating_body(x_ref, wg_ref, bg_ref, w_ref, idx_ref):
    logits = jnp.dot(x_ref[...], wg_ref[...],
                     preferred_element_type=jnp.float32,
                     precision=lax.Precision.HIGHEST) + bg_ref[...]
    m = jnp.max(logits, axis=1, keepdims=True)
    p = jnp.exp(logits - m)
    p = p / jnp.sum(p, axis=1, keepdims=True)
    lane = lax.broadcasted_iota(jnp.int32, p.shape, 1)
    m1 = jnp.max(p, axis=1, keepdims=True)
    i1 = jnp.min(jnp.where(p == m1, lane, LANES), axis=1, keepdims=True)
    p2 = jnp.where(lane == i1, -1.0, p)
    m2 = jnp.max(p2, axis=1, keepdims=True)
    i2 = jnp.min(jnp.where(p2 == m2, lane, LANES), axis=1, keepdims=True)
    w_ref[...] = jnp.concatenate([m1, m2], axis=1)
    idx_ref[...] = jnp.concatenate([i1, i2], axis=1)


def _gating(x, wg_pad, bg_pad):
    bt = 512
    return pl.pallas_call(
        _gating_body,
        grid=(T // bt,),
        in_specs=[
            pl.BlockSpec((bt, D), lambda i: (i, 0)),
            pl.BlockSpec((D, LANES), lambda i: (0, 0)),
            pl.BlockSpec((1, LANES), lambda i: (0, 0)),
        ],
        out_specs=[
            pl.BlockSpec((bt, K), lambda i: (i, 0)),
            pl.BlockSpec((bt, K), lambda i: (i, 0)),
        ],
        out_shape=[
            jax.ShapeDtypeStruct((T, K), jnp.float32),
            jax.ShapeDtypeStruct((T, K), jnp.int32),
        ],
    )(x, wg_pad, bg_pad)


_GCH = 64


def _sc_gather_body(x_hbm, idx_hbm, xs_hbm, idxv, rows, sem):
    wid = lax.axis_index("s") * NC + lax.axis_index("c")
    rpw = NMAX // NW
    base = pl.multiple_of(wid * rpw, _GCH)

    def chunk(c, carry):
        b = pl.multiple_of(base + c * _GCH, _GCH)
        pltpu.sync_copy(idx_hbm.at[pl.ds(b, _GCH)], idxv)
        pltpu.async_copy(x_hbm.at[idxv], rows, sem).wait()
        pltpu.sync_copy(rows, xs_hbm.at[pl.ds(b, _GCH)])
        return carry

    lax.fori_loop(0, rpw // _GCH, chunk, 0)


def _sc_gather(x, src_row):
    return pl.kernel(
        _sc_gather_body,
        out_type=jax.ShapeDtypeStruct((NMAX, D), jnp.float32),
        mesh=plsc.VectorSubcoreMesh(core_axis_name="c", subcore_axis_name="s"),
        scratch_types=[
            pltpu.VMEM((_GCH,), jnp.int32),
            pltpu.VMEM((_GCH, D), jnp.float32),
            pltpu.SemaphoreType.DMA,
        ],
    )(x, src_row)


def _ffn_body(bexp_ref, valid_ref, xs_ref, w1_ref, b1_ref, w2_ref, b2_ref,
              wgt_ref, y_ref):
    i = pl.program_id(0)

    @pl.when(valid_ref[i] > 0)
    def _():
        xb = xs_ref[...].astype(jnp.bfloat16)
        h = jnp.dot(xb, w1_ref[0], preferred_element_type=jnp.float32)
        h = jnp.maximum(h + b1_ref[0], 0.0).astype(jnp.bfloat16)
        y = jnp.dot(h, w2_ref[0], preferred_element_type=jnp.float32)
        y_ref[...] = (y + b2_ref[0]) * wgt_ref[...]


def _ffn(bexp, valid, xs, w1, b1, w2, b2, wgt):
    grid_spec = pltpu.PrefetchScalarGridSpec(
        num_scalar_prefetch=2,
        grid=(NB,),
        in_specs=[
            pl.BlockSpec((B, D), lambda i, be, va: (i, 0)),
            pl.BlockSpec((1, D, DFF), lambda i, be, va: (be[i], 0, 0)),
            pl.BlockSpec((1, 1, DFF), lambda i, be, va: (be[i], 0, 0)),
            pl.BlockSpec((1, DFF, OUT), lambda i, be, va: (be[i], 0, 0)),
            pl.BlockSpec((1, 1, OUT), lambda i, be, va: (be[i], 0, 0)),
            pl.BlockSpec((B, 1), lambda i, be, va: (i, 0)),
        ],
        out_specs=pl.BlockSpec((B, OUT), lambda i, be, va: (i, 0)),
    )
    return pl.pallas_call(
        _ffn_body,
        grid_spec=grid_spec,
        out_shape=jax.ShapeDtypeStruct((NMAX, OUT), jnp.float32),
        compiler_params=pltpu.CompilerParams(
            dimension_semantics=("arbitrary",),
        ),
    )(bexp, valid, xs, w1, b1, w2, b2, wgt)


_CCH = 32


def _sc_combine_body(yw_hbm, s0_hbm, s1_hbm, out_hbm, i0v, i1v, r0, r1,
                     sem0, sem1):
    wid = lax.axis_index("s") * NC + lax.axis_index("c")
    tpw = T // NW
    base = pl.multiple_of(wid * tpw, _CCH)

    def chunk(c, carry):
        b = pl.multiple_of(base + c * _CCH, _CCH)
        pltpu.sync_copy(s0_hbm.at[pl.ds(b, _CCH)], i0v)
        pltpu.sync_copy(s1_hbm.at[pl.ds(b, _CCH)], i1v)
        cp0 = pltpu.async_copy(yw_hbm.at[i0v], r0, sem0)
        cp1 = pltpu.async_copy(yw_hbm.at[i1v], r1, sem1)
        cp0.wait()
        cp1.wait()

        def row(rr, cc):
            for j in range(OUT // 16):
                sl = pl.ds(j * 16, 16)
                r0[rr, sl] = r0[rr, sl] + r1[rr, sl]
            return cc

        lax.fori_loop(0, _CCH, row, 0)
        pltpu.sync_copy(r0, out_hbm.at[pl.ds(b, _CCH)])
        return carry

    lax.fori_loop(0, tpw // _CCH, chunk, 0)


def _sc_combine(yw, s0, s1):
    return pl.kernel(
        _sc_combine_body,
        out_type=jax.ShapeDtypeStruct((T, OUT), jnp.float32),
        mesh=plsc.VectorSubcoreMesh(core_axis_name="c", subcore_axis_name="s"),
        scratch_types=[
            pltpu.VMEM((_CCH,), jnp.int32),
            pltpu.VMEM((_CCH,), jnp.int32),
            pltpu.VMEM((_CCH, OUT), jnp.float32),
            pltpu.VMEM((_CCH, OUT), jnp.float32),
            pltpu.SemaphoreType.DMA,
            pltpu.SemaphoreType.DMA,
        ],
    )(yw, s0, s1)


def _route_indices(idx, w):
    e_flat = idx.reshape(-1)
    oh = (e_flat[:, None] == jnp.arange(E)[None, :]).astype(jnp.int32)
    cum = jnp.cumsum(oh, axis=0)
    rank = jnp.take_along_axis(cum, e_flat[:, None], axis=1)[:, 0] - 1
    counts = cum[-1]
    nblk = (counts + B - 1) // B
    cblk = jnp.cumsum(nblk)
    starts = (cblk - nblk) * B
    dest = starts[e_flat] + rank
    slots = jnp.arange(T * K, dtype=jnp.int32)
    src_row = jnp.zeros((NMAX,), jnp.int32).at[dest].set(slots // K)
    wgt = jnp.zeros((NMAX,), jnp.float32).at[dest].set(w.reshape(-1))
    used = cblk[E - 1]
    bi = jnp.arange(NB, dtype=jnp.int32)
    valid = (bi < used).astype(jnp.int32)
    bi_c = jnp.minimum(bi, used - 1)
    bexp = jnp.searchsorted(cblk, bi_c, side="right").astype(jnp.int32)
    sl = dest.reshape(T, K)
    return src_row, wgt.reshape(NMAX, 1), bexp, valid, sl[:, 0], sl[:, 1]


@jax.jit
def kernel(x, Wg, bg, W1, b1, W2, b2):
    logits = x @ Wg + bg
    probs = jax.nn.softmax(logits, axis=-1)
    w, idx = lax.top_k(probs, K)
    src_row, wgt, bexp, valid, s0, s1 = _route_indices(idx, w)
    xs = _sc_gather(x, src_row)
    yw = _ffn(bexp, valid, xs,
              W1.astype(jnp.bfloat16), b1.reshape(E, 1, DFF),
              W2.astype(jnp.bfloat16), b2.reshape(E, 1, OUT), wgt)
    return _sc_combine(yw, s0, s1)
